# baseline (speedup 1.0000x reference)
import functools

import jax
import jax.numpy as jnp
from jax import lax
from jax.experimental import pallas as pl
from jax.experimental.pallas import tpu as pltpu
from jax.experimental.pallas import tpu_sc as plsc

N_NODES = 10000
D = 128
N_EDGES = 320000

NC = 2
NS = 16
NW = NC * NS

CHUNK = 128
NCH = 80
EPW = NCH * CHUNK
E_PAD = EPW * NW
TOT_CH = E_PAD // CHUNK
K0 = TOT_CH // NS
IBLK = 32
BIN = N_NODES
ACC_ROWS = 10240
DEG_W = 16
DEG_ROWS = 10240

@functools.cache
def _get_deg_kernel():
    mesh = plsc.VectorSubcoreMesh(
        core_axis_name="c", subcore_axis_name="s",
        num_cores=NC, num_subcores=NS)
    return pl.kernel(
        _deg_body,
        out_type=jax.ShapeDtypeStruct((NW, 1, DEG_ROWS), jnp.float32),
        mesh=mesh,
        scratch_types=[
            pltpu.VMEM((NCH, CHUNK), jnp.int32),
            pltpu.VMEM((DEG_ROWS,), jnp.float32)
        ],
        compiler_params=pltpu.CompilerParams(needs_layout_passes=False),
    )


def _deg_body(z_hbm, dst_hbm, out_hbm, didx, hist):
    c = lax.axis_index("c")
    s = lax.axis_index("s")
    wid = c * NS + s

    pltpu.sync_copy(z_hbm, hist)
    pltpu.sync_copy(dst_hbm.at[wid], didx)
    ones = jnp.ones((16,), jnp.float32)

    def chunk_body(k, _):
        for j in range(CHUNK // 16):
            idx16 = didx[k, pl.ds(j * 16, 16)]
            plsc.addupdate_scatter(hist, [idx16], ones)
        return 0

    lax.fori_loop(0, NCH, chunk_body, 0)
    pltpu.sync_copy(hist, out_hbm.at[wid, 0])


@functools.cache
def _get_scatter_kernel():
    mesh = plsc.VectorSubcoreMesh(
        core_axis_name="c", subcore_axis_name="s",
        num_cores=NC, num_subcores=NS)
    return pl.kernel(
        _scatter_body,
        out_type=jax.ShapeDtypeStruct((ACC_ROWS, D), jnp.float32),
        mesh=mesh,
        scratch_types=[
            pltpu.VMEM((IBLK, CHUNK), jnp.int32),
            pltpu.VMEM((IBLK, CHUNK), jnp.int32),
            pltpu.VMEM((2, CHUNK, D), jnp.float32),
            pltpu.VMEM_SHARED((ACC_ROWS, D), jnp.float32),
            pltpu.SemaphoreType.DMA,
            pltpu.SemaphoreType.DMA,
        ],
    )


def _scatter_body(src_hbm, dst_hbm, g_hbm, z_hbm, out_hbm,
                  sidx, didx, rows, acc, sem0, sem1):
    c = lax.axis_index("c")
    s = lax.axis_index("s")
    sems = (sem0, sem1)

    @pl.when(c == 0)
    def _():
        zrows = ACC_ROWS // NS
        pltpu.sync_copy(z_hbm, acc.at[pl.ds(s * zrows, zrows)])
        plsc.subcore_barrier()

        base = s * K0

        def issue(k, b):
            pltpu.async_copy(g_hbm.at[sidx.at[k]], rows.at[b], sems[b])

        def drain_and_scatter(k, b):
            pltpu.make_async_copy(
                g_hbm.at[sidx.at[k]], rows.at[b], sems[b]).wait()
            pltpu.sync_copy(rows.at[b], acc.at[didx.at[k]], add=True)

        def blk_body(j, _):
            off = base + j * IBLK
            pltpu.sync_copy(src_hbm.at[pl.ds(off, IBLK)], sidx)
            pltpu.sync_copy(dst_hbm.at[pl.ds(off, IBLK)], didx)
            issue(0, 0)

            def pair_body(i, _):
                issue(2 * i + 1, 1)
                drain_and_scatter(2 * i, 0)

                @pl.when(i < IBLK // 2 - 1)
                def _():
                    issue(2 * i + 2, 0)

                drain_and_scatter(2 * i + 1, 1)
                return 0

            lax.fori_loop(0, IBLK // 2, pair_body, 0)
            return 0

        lax.fori_loop(0, K0 // IBLK, blk_body, 0)
        plsc.subcore_barrier()

        orows = ACC_ROWS // NS
        pltpu.sync_copy(acc.at[pl.ds(s * orows, orows)],
                        out_hbm.at[pl.ds(s * orows, orows)])


ROWS_BLK = 400
GRID = N_NODES // ROWS_BLK


def _mm_scale_body(x_ref, w_ref, dinv_ref, o_ref):
    h = jnp.dot(x_ref[...], w_ref[...], preferred_element_type=jnp.float32)
    o_ref[...] = h * dinv_ref[...]


def _combine_mm_body(a_ref, g_ref, dinv_ref, b_ref, w_ref, o_ref):
    a = a_ref[...] + g_ref[...]
    y = jnp.maximum(a * dinv_ref[...] + b_ref[...], 0.0)
    h = jnp.dot(y, w_ref[...], preferred_element_type=jnp.float32)
    o_ref[...] = h * dinv_ref[...]


def _combine_body(a_ref, g_ref, dinv_ref, b_ref, o_ref):
    a = a_ref[...] + g_ref[...]
    o_ref[...] = a * dinv_ref[...] + b_ref[...]


def _blk(shape, imap):
    return pl.BlockSpec(shape, imap)


_row_spec = _blk((ROWS_BLK, D), lambda i: (i, 0))
_dinv_spec = _blk((ROWS_BLK, 1), lambda i: (i, 0))
_w_spec = _blk((D, D), lambda i: (0, 0))
_b_spec = _blk((1, D), lambda i: (0, 0))
_acc_spec = _blk((ROWS_BLK, D), lambda i: (i, 0))
_out_sds = jax.ShapeDtypeStruct((N_NODES, D), jnp.float32)

_mm_scale = pl.pallas_call(
    _mm_scale_body,
    grid=(GRID,),
    in_specs=[_row_spec, _w_spec, _dinv_spec],
    out_specs=_row_spec,
    out_shape=_out_sds,
)

_combine_mm = pl.pallas_call(
    _combine_mm_body,
    grid=(GRID,),
    in_specs=[_acc_spec, _row_spec, _dinv_spec, _b_spec, _w_spec],
    out_specs=_row_spec,
    out_shape=_out_sds,
)

_combine = pl.pallas_call(
    _combine_body,
    grid=(GRID,),
    in_specs=[_acc_spec, _row_spec, _dinv_spec, _b_spec],
    out_specs=_row_spec,
    out_shape=_out_sds,
)


def kernel(x, edge_attr, edge_index, W1, b1, W2, b2):
    del edge_attr
    src = edge_index[0].astype(jnp.int32)
    dst = edge_index[1].astype(jnp.int32)
    pad = E_PAD - N_EDGES
    pad_dst = BIN + (jnp.arange(pad, dtype=jnp.int32) % (ACC_ROWS - N_NODES))
    src_f = jnp.concatenate(
        [src, jnp.zeros((pad,), jnp.int32)]).reshape(TOT_CH, CHUNK)
    dst_f = jnp.concatenate([dst, pad_dst]).reshape(TOT_CH, CHUNK)
    dst_p = dst_f.reshape(NW, NCH, CHUNK)
    zeros_tile = jnp.zeros((ACC_ROWS // NS, D), jnp.float32)

    zdeg = jnp.zeros((DEG_ROWS,), jnp.float32)
    degp = _get_deg_kernel()(zdeg, dst_p)
    deg = degp[:, 0, :N_NODES].sum(axis=0) + 1.0
    dinv = lax.rsqrt(deg)[:, None]

    b1r = b1[None, :]
    b2r = b2[None, :]

    g1 = _mm_scale(x, W1, dinv)
    acc1 = _get_scatter_kernel()(src_f, dst_f, g1, zeros_tile)
    g2 = _combine_mm(acc1, g1, dinv, b1r, W2)
    acc2 = _get_scatter_kernel()(src_f, dst_f, g2, zeros_tile)
    return _combine(acc2, g2, dinv, b2r)


# speedup vs baseline: 1.0029x; 1.0029x over previous
import functools

import jax
import jax.numpy as jnp
from jax import lax
from jax.experimental import pallas as pl
from jax.experimental.pallas import tpu as pltpu
from jax.experimental.pallas import tpu_sc as plsc

N_NODES = 10000
D = 128
N_EDGES = 320000

NC = 2
NS = 16
NW = NC * NS

CHUNK = 64
NCH = 160
EPW = NCH * CHUNK
E_PAD = EPW * NW
TOT_CH = E_PAD // CHUNK
K0 = 256
K1 = TOT_CH // NS - K0
IBLK = 64
NBUF = 4
BIN = N_NODES
ACC_ROWS = 10240
DEG_W = 16
DEG_ROWS = 10240

@functools.cache
def _get_deg_kernel():
    mesh = plsc.VectorSubcoreMesh(
        core_axis_name="c", subcore_axis_name="s",
        num_cores=NC, num_subcores=NS)
    return pl.kernel(
        _deg_body,
        out_type=jax.ShapeDtypeStruct((NW, 1, DEG_ROWS), jnp.float32),
        mesh=mesh,
        scratch_types=[
            pltpu.VMEM((NCH, CHUNK), jnp.int32),
            pltpu.VMEM((DEG_ROWS,), jnp.float32)
        ],
        compiler_params=pltpu.CompilerParams(needs_layout_passes=False),
    )


def _deg_body(z_hbm, dst_hbm, out_hbm, didx, hist):
    c = lax.axis_index("c")
    s = lax.axis_index("s")
    wid = c * NS + s

    pltpu.sync_copy(z_hbm, hist)
    pltpu.sync_copy(dst_hbm.at[wid], didx)
    ones = jnp.ones((16,), jnp.float32)

    def chunk_body(k, _):
        for j in range(CHUNK // 16):
            idx16 = didx[k, pl.ds(j * 16, 16)]
            plsc.addupdate_scatter(hist, [idx16], ones)
        return 0

    lax.fori_loop(0, NCH, chunk_body, 0)
    pltpu.sync_copy(hist, out_hbm.at[wid, 0])


@functools.cache
def _get_scatter_kernel():
    mesh = plsc.VectorSubcoreMesh(
        core_axis_name="c", subcore_axis_name="s",
        num_cores=NC, num_subcores=NS)
    return pl.kernel(
        _scatter_body,
        out_type=jax.ShapeDtypeStruct((NC, ACC_ROWS, D), jnp.float32),
        mesh=mesh,
        scratch_types=[
            pltpu.VMEM((IBLK, CHUNK), jnp.int32),
            pltpu.VMEM((IBLK, CHUNK), jnp.int32),
            pltpu.VMEM((NBUF, CHUNK, D), jnp.float32),
            pltpu.VMEM_SHARED((ACC_ROWS, D), jnp.float32),
            pltpu.SemaphoreType.DMA,
            pltpu.SemaphoreType.DMA,
            pltpu.SemaphoreType.DMA,
            pltpu.SemaphoreType.DMA,
        ],
    )


def _scatter_body(src_hbm, dst_hbm, g_hbm, z_hbm, out_hbm,
                  sidx, didx, rows, acc, sem0, sem1, sem2, sem3):
    c = lax.axis_index("c")
    s = lax.axis_index("s")
    sems = (sem0, sem1, sem2, sem3)

    zrows = ACC_ROWS // NS
    pltpu.sync_copy(z_hbm, acc.at[pl.ds(s * zrows, zrows)])
    plsc.subcore_barrier()

    base = jnp.where(c == 0, s * K0, NS * K0 + s * K1)
    nblk = jnp.where(c == 0, K0 // IBLK, K1 // IBLK)

    def issue(k, b):
        pltpu.async_copy(g_hbm.at[sidx.at[k]], rows.at[b], sems[b])

    def drain_and_scatter(k, b):
        pltpu.make_async_copy(g_hbm.at[sidx.at[k]], rows.at[b], sems[b]).wait()
        pltpu.sync_copy(rows.at[b], acc.at[didx.at[k]], add=True)

    def blk_body(j, _):
        @pl.when(j < nblk)
        def _():
            off = base + j * IBLK
            pltpu.sync_copy(src_hbm.at[pl.ds(off, IBLK)], sidx)
            pltpu.sync_copy(dst_hbm.at[pl.ds(off, IBLK)], didx)
            for b in range(NBUF):
                issue(b, b)

            def quad_body(i, _):
                for b in range(NBUF):
                    k = NBUF * i + b
                    drain_and_scatter(k, b)

                    @pl.when(k + NBUF < IBLK)
                    def _():
                        issue(k + NBUF, b)
                return 0

            lax.fori_loop(0, IBLK // NBUF, quad_body, 0)
        return 0

    lax.fori_loop(0, K0 // IBLK, blk_body, 0)
    plsc.subcore_barrier()

    orows = ACC_ROWS // NS
    pltpu.sync_copy(acc.at[pl.ds(s * orows, orows)],
                    out_hbm.at[c, pl.ds(s * orows, orows)])


ROWS_BLK = 400
GRID = N_NODES // ROWS_BLK


def _mm_scale_body(x_ref, w_ref, dinv_ref, o_ref):
    h = jnp.dot(x_ref[...], w_ref[...], preferred_element_type=jnp.float32)
    o_ref[...] = h * dinv_ref[...]


def _combine_mm_body(a_ref, g_ref, dinv_ref, b_ref, w_ref, o_ref):
    a = a_ref[0] + a_ref[1] + g_ref[...]
    y = jnp.maximum(a * dinv_ref[...] + b_ref[...], 0.0)
    h = jnp.dot(y, w_ref[...], preferred_element_type=jnp.float32)
    o_ref[...] = h * dinv_ref[...]


def _combine_body(a_ref, g_ref, dinv_ref, b_ref, o_ref):
    a = a_ref[0] + a_ref[1] + g_ref[...]
    o_ref[...] = a * dinv_ref[...] + b_ref[...]


def _blk(shape, imap):
    return pl.BlockSpec(shape, imap)


_row_spec = _blk((ROWS_BLK, D), lambda i: (i, 0))
_dinv_spec = _blk((ROWS_BLK, 1), lambda i: (i, 0))
_w_spec = _blk((D, D), lambda i: (0, 0))
_b_spec = _blk((1, D), lambda i: (0, 0))
_acc_spec = _blk((NC, ROWS_BLK, D), lambda i: (0, i, 0))
_out_sds = jax.ShapeDtypeStruct((N_NODES, D), jnp.float32)

_mm_scale = pl.pallas_call(
    _mm_scale_body,
    grid=(GRID,),
    in_specs=[_row_spec, _w_spec, _dinv_spec],
    out_specs=_row_spec,
    out_shape=_out_sds,
)

_combine_mm = pl.pallas_call(
    _combine_mm_body,
    grid=(GRID,),
    in_specs=[_acc_spec, _row_spec, _dinv_spec, _b_spec, _w_spec],
    out_specs=_row_spec,
    out_shape=_out_sds,
)

_combine = pl.pallas_call(
    _combine_body,
    grid=(GRID,),
    in_specs=[_acc_spec, _row_spec, _dinv_spec, _b_spec],
    out_specs=_row_spec,
    out_shape=_out_sds,
)


def kernel(x, edge_attr, edge_index, W1, b1, W2, b2):
    del edge_attr
    src = edge_index[0].astype(jnp.int32)
    dst = edge_index[1].astype(jnp.int32)
    pad = E_PAD - N_EDGES
    pad_dst = BIN + (jnp.arange(pad, dtype=jnp.int32) % (ACC_ROWS - N_NODES))
    src_f = jnp.concatenate(
        [src, jnp.zeros((pad,), jnp.int32)]).reshape(TOT_CH, CHUNK)
    dst_f = jnp.concatenate([dst, pad_dst]).reshape(TOT_CH, CHUNK)
    dst_p = dst_f.reshape(NW, NCH, CHUNK)
    zeros_tile = jnp.zeros((ACC_ROWS // NS, D), jnp.float32)

    zdeg = jnp.zeros((DEG_ROWS,), jnp.float32)
    degp = _get_deg_kernel()(zdeg, dst_p)
    deg = degp[:, 0, :N_NODES].sum(axis=0) + 1.0
    dinv = lax.rsqrt(deg)[:, None]

    b1r = b1[None, :]
    b2r = b2[None, :]

    g1 = _mm_scale(x, W1, dinv)
    acc1 = _get_scatter_kernel()(src_f, dst_f, g1, zeros_tile)
    g2 = _combine_mm(acc1, g1, dinv, b1r, W2)
    acc2 = _get_scatter_kernel()(src_f, dst_f, g2, zeros_tile)
    return _combine(acc2, g2, dinv, b2r)


# speedup vs baseline: 1.1202x; 1.1170x over previous
import functools

import jax
import jax.numpy as jnp
from jax import lax
from jax.experimental import pallas as pl
from jax.experimental.pallas import tpu as pltpu
from jax.experimental.pallas import tpu_sc as plsc

N_NODES = 10000
D = 128
N_EDGES = 320000

NC = 2
NS = 16
NW = NC * NS

CHUNK = 128
NCH = 80
EPW = NCH * CHUNK
E_PAD = EPW * NW
TOT_CH = E_PAD // CHUNK
K0 = 128
K1 = TOT_CH // NS - K0
IBLK = 32
NBUF = 2
BIN = N_NODES
ACC_ROWS = 10240
DEG_W = 16
DEG_ROWS = 10240

@functools.cache
def _get_deg_kernel():
    mesh = plsc.VectorSubcoreMesh(
        core_axis_name="c", subcore_axis_name="s",
        num_cores=NC, num_subcores=NS)
    return pl.kernel(
        _deg_body,
        out_type=jax.ShapeDtypeStruct((NW, 1, DEG_ROWS), jnp.float32),
        mesh=mesh,
        scratch_types=[
            pltpu.VMEM((NCH, CHUNK), jnp.int32),
            pltpu.VMEM((DEG_ROWS,), jnp.float32)
        ],
        compiler_params=pltpu.CompilerParams(needs_layout_passes=False),
    )


def _deg_body(z_hbm, dst_hbm, out_hbm, didx, hist):
    c = lax.axis_index("c")
    s = lax.axis_index("s")
    wid = c * NS + s

    pltpu.sync_copy(z_hbm, hist)
    pltpu.sync_copy(dst_hbm.at[wid], didx)
    ones = jnp.ones((16,), jnp.float32)

    def chunk_body(k, _):
        for j in range(CHUNK // 16):
            idx16 = didx[k, pl.ds(j * 16, 16)]
            plsc.addupdate_scatter(hist, [idx16], ones)
        return 0

    lax.fori_loop(0, NCH, chunk_body, 0)
    pltpu.sync_copy(hist, out_hbm.at[wid, 0])


@functools.cache
def _get_scatter_kernel():
    mesh = plsc.VectorSubcoreMesh(
        core_axis_name="c", subcore_axis_name="s",
        num_cores=NC, num_subcores=NS)
    return pl.kernel(
        _scatter_body,
        out_type=jax.ShapeDtypeStruct((NC, ACC_ROWS, D), jnp.float32),
        mesh=mesh,
        scratch_types=[
            pltpu.VMEM((IBLK, CHUNK), jnp.int32),
            pltpu.VMEM((IBLK, CHUNK), jnp.int32),
            pltpu.VMEM((NBUF, CHUNK, D), jnp.float32),
            pltpu.VMEM_SHARED((ACC_ROWS, D), jnp.float32),
            pltpu.SemaphoreType.DMA,
            pltpu.SemaphoreType.DMA,
            pltpu.SemaphoreType.DMA,
            pltpu.SemaphoreType.DMA,
        ],
    )


def _scatter_body(src_hbm, dst_hbm, g_hbm, z_hbm, out_hbm,
                  sidx, didx, rows, acc, sem0, sem1, sem2, sem3):
    c = lax.axis_index("c")
    s = lax.axis_index("s")
    sems = (sem0, sem1, sem2, sem3)

    zrows = ACC_ROWS // NS
    pltpu.sync_copy(z_hbm, acc.at[pl.ds(s * zrows, zrows)])
    plsc.subcore_barrier()

    base = jnp.where(c == 0, s * K0, NS * K0 + s * K1)
    nblk = jnp.where(c == 0, K0 // IBLK, K1 // IBLK)

    def issue(k, b):
        pltpu.async_copy(g_hbm.at[sidx.at[k]], rows.at[b], sems[b])

    def drain_and_scatter(k, b):
        pltpu.make_async_copy(g_hbm.at[sidx.at[k]], rows.at[b], sems[b]).wait()
        pltpu.sync_copy(rows.at[b], acc.at[didx.at[k]], add=True)

    def blk_body(j, _):
        @pl.when(j < nblk)
        def _():
            off = base + j * IBLK
            pltpu.sync_copy(src_hbm.at[pl.ds(off, IBLK)], sidx)
            pltpu.sync_copy(dst_hbm.at[pl.ds(off, IBLK)], didx)
            for b in range(NBUF):
                issue(b, b)

            def quad_body(i, _):
                for b in range(NBUF):
                    k = NBUF * i + b
                    drain_and_scatter(k, b)

                    @pl.when(k + NBUF < IBLK)
                    def _():
                        issue(k + NBUF, b)
                return 0

            lax.fori_loop(0, IBLK // NBUF, quad_body, 0)
        return 0

    lax.fori_loop(0, K0 // IBLK, blk_body, 0)
    plsc.subcore_barrier()

    orows = ACC_ROWS // NS
    pltpu.sync_copy(acc.at[pl.ds(s * orows, orows)],
                    out_hbm.at[c, pl.ds(s * orows, orows)])


ROWS_BLK = 400
GRID = N_NODES // ROWS_BLK


def _mm_scale_body(x_ref, w_ref, dinv_ref, o_ref):
    h = jnp.dot(x_ref[...], w_ref[...], preferred_element_type=jnp.float32)
    o_ref[...] = h * dinv_ref[...]


def _combine_mm_body(a_ref, g_ref, dinv_ref, b_ref, w_ref, o_ref):
    a = a_ref[0] + a_ref[1] + g_ref[...]
    y = jnp.maximum(a * dinv_ref[...] + b_ref[...], 0.0)
    h = jnp.dot(y, w_ref[...], preferred_element_type=jnp.float32)
    o_ref[...] = h * dinv_ref[...]


def _combine_body(a_ref, g_ref, dinv_ref, b_ref, o_ref):
    a = a_ref[0] + a_ref[1] + g_ref[...]
    o_ref[...] = a * dinv_ref[...] + b_ref[...]


def _blk(shape, imap):
    return pl.BlockSpec(shape, imap)


_row_spec = _blk((ROWS_BLK, D), lambda i: (i, 0))
_dinv_spec = _blk((ROWS_BLK, 1), lambda i: (i, 0))
_w_spec = _blk((D, D), lambda i: (0, 0))
_b_spec = _blk((1, D), lambda i: (0, 0))
_acc_spec = _blk((NC, ROWS_BLK, D), lambda i: (0, i, 0))
_out_sds = jax.ShapeDtypeStruct((N_NODES, D), jnp.float32)

_mm_scale = pl.pallas_call(
    _mm_scale_body,
    grid=(GRID,),
    in_specs=[_row_spec, _w_spec, _dinv_spec],
    out_specs=_row_spec,
    out_shape=_out_sds,
)

_combine_mm = pl.pallas_call(
    _combine_mm_body,
    grid=(GRID,),
    in_specs=[_acc_spec, _row_spec, _dinv_spec, _b_spec, _w_spec],
    out_specs=_row_spec,
    out_shape=_out_sds,
)

_combine = pl.pallas_call(
    _combine_body,
    grid=(GRID,),
    in_specs=[_acc_spec, _row_spec, _dinv_spec, _b_spec],
    out_specs=_row_spec,
    out_shape=_out_sds,
)


def kernel(x, edge_attr, edge_index, W1, b1, W2, b2):
    del edge_attr
    src = edge_index[0].astype(jnp.int32)
    dst = edge_index[1].astype(jnp.int32)
    pad = E_PAD - N_EDGES
    pad_dst = BIN + (jnp.arange(pad, dtype=jnp.int32) % (ACC_ROWS - N_NODES))
    src_f = jnp.concatenate(
        [src, jnp.zeros((pad,), jnp.int32)]).reshape(TOT_CH, CHUNK)
    dst_f = jnp.concatenate([dst, pad_dst]).reshape(TOT_CH, CHUNK)
    dst_p = dst_f.reshape(NW, NCH, CHUNK)
    zeros_tile = jnp.zeros((ACC_ROWS // NS, D), jnp.float32)

    zdeg = jnp.zeros((DEG_ROWS,), jnp.float32)
    degp = _get_deg_kernel()(zdeg, dst_p)
    deg = degp[:, 0, :N_NODES].sum(axis=0) + 1.0
    dinv = lax.rsqrt(deg)[:, None]

    b1r = b1[None, :]
    b2r = b2[None, :]

    g1 = _mm_scale(x, W1, dinv)
    acc1 = _get_scatter_kernel()(src_f, dst_f, g1, zeros_tile)
    g2 = _combine_mm(acc1, g1, dinv, b1r, W2)
    acc2 = _get_scatter_kernel()(src_f, dst_f, g2, zeros_tile)
    return _combine(acc2, g2, dinv, b2r)
